# baseline (speedup 1.0000x reference)
import functools
import math

import jax
import jax.numpy as jnp
from jax import lax
from jax.experimental import pallas as pl
from jax.experimental.pallas import tpu as pltpu
from jax.experimental.pallas import tpu_sc as plsc

D_MODEL = 768
LANES = 16
NUM_CORES = 2
NUM_SUBCORES = 16
NUM_WORKERS = NUM_CORES * NUM_SUBCORES
CHUNK = 64
SCALE = 1.0 / math.sqrt(D_MODEL)


def _sc_body(word_hbm, task_hbm, seg_hbm, tid_hbm, sid_hbm, ids_hbm, out_hbm,
             small_v, comb_v, ids_v, tid_v, sid_v, cidx_v, buf, obuf, sem):
    n_tok = ids_hbm.shape[0]
    tok_per_w = n_tok // NUM_WORKERS
    wid = lax.axis_index("s") * NUM_CORES + lax.axis_index("c")
    base = wid * tok_per_w

    pltpu.sync_copy(task_hbm, small_v.at[pl.ds(0, 3)])
    pltpu.sync_copy(seg_hbm, small_v.at[pl.ds(3, 3)])

    @pl.loop(0, D_MODEL // LANES)
    def _build(j):
        col = j * LANES
        for t in range(3):
            tv = small_v[t, pl.ds(col, LANES)]
            for s in range(3):
                sv = small_v[3 + s, pl.ds(col, LANES)]
                comb_v[pl.ds((t * 3 + s) * D_MODEL + col, LANES)] = tv + sv * SCALE

    lane_iota = lax.iota(jnp.int32, LANES)

    for ch in range(tok_per_w // CHUNK):
        off = base + ch * CHUNK
        pltpu.sync_copy(ids_hbm.at[pl.ds(off, CHUNK)], ids_v)
        pltpu.sync_copy(tid_hbm.at[pl.ds(off, CHUNK)], tid_v)
        pltpu.sync_copy(sid_hbm.at[pl.ds(off, CHUNK)], sid_v)
        for g in range(CHUNK // LANES):
            sl = pl.ds(g * LANES, LANES)
            cidx_v[sl] = tid_v[sl] * 3 + sid_v[sl]
        pltpu.async_copy(word_hbm.at[ids_v], buf, sem).wait()

        @pl.loop(0, CHUNK)
        def _add(t, ch=ch):
            ctv = plsc.load_gather(cidx_v, [jnp.full((LANES,), 0, jnp.int32) + t])
            cbase = ctv * D_MODEL + lane_iota
            for j in range(D_MODEL // LANES):
                cvals = plsc.load_gather(comb_v, [cbase + (j * LANES)])
                csl = pl.ds(j * LANES, LANES)
                obuf[t, csl] = buf[t, csl] + cvals

        pltpu.sync_copy(obuf, out_hbm.at[pl.ds(off, CHUNK)])


@functools.lru_cache(maxsize=None)
def _make_sc_call(n_tok: int):
    return pl.kernel(
        _sc_body,
        out_type=jax.ShapeDtypeStruct((n_tok, D_MODEL), jnp.float32),
        mesh=plsc.VectorSubcoreMesh(core_axis_name="c", subcore_axis_name="s"),
        compiler_params=pltpu.CompilerParams(
            use_tc_tiling_on_sc=False, needs_layout_passes=False),
        scratch_types=[
            pltpu.VMEM((6, D_MODEL), jnp.float32),
            pltpu.VMEM((9 * D_MODEL,), jnp.float32),
            pltpu.VMEM((CHUNK,), jnp.int32),
            pltpu.VMEM((CHUNK,), jnp.int32),
            pltpu.VMEM((CHUNK,), jnp.int32),
            pltpu.VMEM((CHUNK,), jnp.int32),
            pltpu.VMEM((CHUNK, D_MODEL), jnp.float32),
            pltpu.VMEM((CHUNK, D_MODEL), jnp.float32),
            pltpu.SemaphoreType.DMA,
        ],
    )


@jax.jit
def kernel(input_ids, task_ids, segment_ids, word_table, task_table, segment_table):
    shape = input_ids.shape
    ids = input_ids.reshape(-1).astype(jnp.int32)
    tid = task_ids.reshape(-1).astype(jnp.int32)
    sid = segment_ids.reshape(-1).astype(jnp.int32)
    out = _make_sc_call(ids.shape[0])(
        word_table, task_table, segment_table, tid, sid, ids)
    return out.reshape(shape + (D_MODEL,))


# speedup vs baseline: 1.3413x; 1.3413x over previous
import functools
import math

import jax
import jax.numpy as jnp
from jax import lax
from jax.experimental import pallas as pl
from jax.experimental.pallas import tpu as pltpu
from jax.experimental.pallas import tpu_sc as plsc

D_MODEL = 768
LANES = 16
NUM_CORES = 2
NUM_SUBCORES = 16
NUM_WORKERS = NUM_CORES * NUM_SUBCORES
CHUNK = 64
SCALE = 1.0 / math.sqrt(D_MODEL)


def _sc_body(word_hbm, task_hbm, seg_hbm, tid_hbm, sid_hbm, ids_hbm, out_hbm,
             small_v, comb_v, ids_v, tid_v, sid_v, cidx_v, buf, sem):
    n_tok = ids_hbm.shape[0]
    tok_per_w = n_tok // NUM_WORKERS
    wid = lax.axis_index("s") * NUM_CORES + lax.axis_index("c")
    base = wid * tok_per_w

    pltpu.sync_copy(task_hbm, small_v.at[pl.ds(0, 3)])
    pltpu.sync_copy(seg_hbm, small_v.at[pl.ds(3, 3)])

    @pl.loop(0, D_MODEL // LANES)
    def _build(j):
        col = j * LANES
        for t in range(3):
            tv = small_v[t, pl.ds(col, LANES)]
            for s in range(3):
                sv = small_v[3 + s, pl.ds(col, LANES)]
                comb_v[pl.ds((t * 3 + s) * D_MODEL + col, LANES)] = tv + sv * SCALE

    lane_iota = lax.iota(jnp.int32, LANES)

    for ch in range(tok_per_w // CHUNK):
        off = base + ch * CHUNK
        pltpu.sync_copy(ids_hbm.at[pl.ds(off, CHUNK)], ids_v)
        pltpu.sync_copy(tid_hbm.at[pl.ds(off, CHUNK)], tid_v)
        pltpu.sync_copy(sid_hbm.at[pl.ds(off, CHUNK)], sid_v)
        for g in range(CHUNK // LANES):
            sl = pl.ds(g * LANES, LANES)
            cidx_v[sl] = tid_v[sl] * 3 + sid_v[sl]
        pltpu.async_copy(word_hbm.at[ids_v], buf, sem).wait()

        @pl.loop(0, CHUNK)
        def _add(t, ch=ch):
            ctv = plsc.load_gather(cidx_v, [jnp.full((LANES,), 0, jnp.int32) + t])
            cbase = ctv * D_MODEL + lane_iota
            nj = D_MODEL // LANES
            depth = 4
            pend = [plsc.load_gather(comb_v, [cbase + (j * LANES)])
                    for j in range(depth)]
            for j in range(nj):
                if j + depth < nj:
                    pend.append(
                        plsc.load_gather(comb_v, [cbase + ((j + depth) * LANES)]))
                plsc.addupdate(buf.at[t, pl.ds(j * LANES, LANES)], pend.pop(0))

        pltpu.sync_copy(buf, out_hbm.at[pl.ds(off, CHUNK)])


@functools.lru_cache(maxsize=None)
def _make_sc_call(n_tok: int):
    return pl.kernel(
        _sc_body,
        out_type=jax.ShapeDtypeStruct((n_tok, D_MODEL), jnp.float32),
        mesh=plsc.VectorSubcoreMesh(core_axis_name="c", subcore_axis_name="s"),
        compiler_params=pltpu.CompilerParams(
            use_tc_tiling_on_sc=False, needs_layout_passes=False),
        scratch_types=[
            pltpu.VMEM((6, D_MODEL), jnp.float32),
            pltpu.VMEM((9 * D_MODEL,), jnp.float32),
            pltpu.VMEM((CHUNK,), jnp.int32),
            pltpu.VMEM((CHUNK,), jnp.int32),
            pltpu.VMEM((CHUNK,), jnp.int32),
            pltpu.VMEM((CHUNK,), jnp.int32),
            pltpu.VMEM((CHUNK, D_MODEL), jnp.float32),
            pltpu.SemaphoreType.DMA,
        ],
    )


@jax.jit
def kernel(input_ids, task_ids, segment_ids, word_table, task_table, segment_table):
    shape = input_ids.shape
    ids = input_ids.reshape(-1).astype(jnp.int32)
    tid = task_ids.reshape(-1).astype(jnp.int32)
    sid = segment_ids.reshape(-1).astype(jnp.int32)
    out = _make_sc_call(ids.shape[0])(
        word_table, task_table, segment_table, tid, sid, ids)
    return out.reshape(shape + (D_MODEL,))


# speedup vs baseline: 1.5655x; 1.1671x over previous
import functools
import math

import jax
import jax.numpy as jnp
from jax import lax
from jax.experimental import pallas as pl
from jax.experimental.pallas import tpu as pltpu
from jax.experimental.pallas import tpu_sc as plsc

D_MODEL = 768
LANES = 16
NUM_CORES = 2
NUM_SUBCORES = 16
NUM_WORKERS = NUM_CORES * NUM_SUBCORES
CHUNK = 32
NBUF = 3
SCALE = 1.0 / math.sqrt(D_MODEL)


def _sc_body(word_hbm, task_hbm, seg_hbm, tid_hbm, sid_hbm, ids_hbm, out_hbm,
             small_v, comb_v, ids_vs, tid_vs, sid_vs, cidx_vs, bufs,
             isems, gsems, osems):
    n_tok = ids_hbm.shape[0]
    tok_per_w = n_tok // NUM_WORKERS
    nch = tok_per_w // CHUNK
    wid = lax.axis_index("s") * NUM_CORES + lax.axis_index("c")
    base = wid * tok_per_w

    pltpu.sync_copy(task_hbm, small_v.at[pl.ds(0, 3)])
    pltpu.sync_copy(seg_hbm, small_v.at[pl.ds(3, 3)])

    @pl.loop(0, D_MODEL // LANES)
    def _build(j):
        col = j * LANES
        for t in range(3):
            tv = small_v[t, pl.ds(col, LANES)]
            for s in range(3):
                sv = small_v[3 + s, pl.ds(col, LANES)]
                comb_v[pl.ds((t * 3 + s) * D_MODEL + col, LANES)] = tv + sv * SCALE

    lane_iota = lax.iota(jnp.int32, LANES)

    def issue_idx(k):
        b = k % NBUF
        off = base + k * CHUNK
        return (
            pltpu.async_copy(ids_hbm.at[pl.ds(off, CHUNK)], ids_vs[b], isems[b]),
            pltpu.async_copy(tid_hbm.at[pl.ds(off, CHUNK)], tid_vs[b], isems[b]),
            pltpu.async_copy(sid_hbm.at[pl.ds(off, CHUNK)], sid_vs[b], isems[b]),
        )

    def start_gather(k, idx_descs, out_desc_old):
        b = k % NBUF
        for d in idx_descs:
            d.wait()
        for g in range(CHUNK // LANES):
            sl = pl.ds(g * LANES, LANES)
            cidx_vs[b][sl] = tid_vs[b][sl] * 3 + sid_vs[b][sl]
        if out_desc_old is not None:
            out_desc_old.wait()
        return pltpu.async_copy(word_hbm.at[ids_vs[b]], bufs[b], gsems[b])

    def finish(k, gather_desc):
        b = k % NBUF
        gather_desc.wait()
        buf = bufs[b]
        cidx_v = cidx_vs[b]

        @pl.loop(0, CHUNK)
        def _add(t):
            ctv = plsc.load_gather(cidx_v, [jnp.full((LANES,), 0, jnp.int32) + t])
            cbase = ctv * D_MODEL + lane_iota
            nj = D_MODEL // LANES
            depth = 4
            pend = [plsc.load_gather(comb_v, [cbase + (j * LANES)])
                    for j in range(depth)]
            for j in range(nj):
                if j + depth < nj:
                    pend.append(
                        plsc.load_gather(comb_v, [cbase + ((j + depth) * LANES)]))
                plsc.addupdate(buf.at[t, pl.ds(j * LANES, LANES)], pend.pop(0))

        off = base + k * CHUNK
        return pltpu.async_copy(buf, out_hbm.at[pl.ds(off, CHUNK)], osems[b])

    idxd = {0: issue_idx(0)}
    if nch > 1:
        idxd[1] = issue_idx(1)
    outd = {}
    gd = {0: start_gather(0, idxd.pop(0), None)}
    for k in range(nch):
        if k + 1 < nch:
            gd[k + 1] = start_gather(k + 1, idxd.pop(k + 1),
                                     outd.get(k + 1 - NBUF))
        if k + 2 < nch:
            idxd[k + 2] = issue_idx(k + 2)
        outd[k] = finish(k, gd.pop(k))
    for k in range(max(0, nch - NBUF + 1), nch):
        outd[k].wait()


@functools.lru_cache(maxsize=None)
def _make_sc_call(n_tok: int):
    def body(word_hbm, task_hbm, seg_hbm, tid_hbm, sid_hbm, ids_hbm, out_hbm,
             small_v, comb_v,
             i0, i1, i2, t0, t1, t2, s0, s1, s2, c0, c1, c2, b0, b1, b2,
             is0, is1, is2, gs0, gs1, gs2, os0, os1, os2):
        _sc_body(word_hbm, task_hbm, seg_hbm, tid_hbm, sid_hbm, ids_hbm,
                 out_hbm, small_v, comb_v,
                 (i0, i1, i2), (t0, t1, t2), (s0, s1, s2), (c0, c1, c2),
                 (b0, b1, b2), (is0, is1, is2), (gs0, gs1, gs2),
                 (os0, os1, os2))

    idx = pltpu.VMEM((CHUNK,), jnp.int32)
    buf = pltpu.VMEM((CHUNK, D_MODEL), jnp.float32)
    sem = pltpu.SemaphoreType.DMA
    return pl.kernel(
        body,
        out_type=jax.ShapeDtypeStruct((n_tok, D_MODEL), jnp.float32),
        mesh=plsc.VectorSubcoreMesh(core_axis_name="c", subcore_axis_name="s"),
        compiler_params=pltpu.CompilerParams(
            use_tc_tiling_on_sc=False, needs_layout_passes=False),
        scratch_types=[
            pltpu.VMEM((6, D_MODEL), jnp.float32),
            pltpu.VMEM((9 * D_MODEL,), jnp.float32),
            idx, idx, idx,
            idx, idx, idx,
            idx, idx, idx,
            idx, idx, idx,
            buf, buf, buf,
            sem, sem, sem,
            sem, sem, sem,
            sem, sem, sem,
        ],
    )


@jax.jit
def kernel(input_ids, task_ids, segment_ids, word_table, task_table, segment_table):
    shape = input_ids.shape
    ids = input_ids.reshape(-1).astype(jnp.int32)
    tid = task_ids.reshape(-1).astype(jnp.int32)
    sid = segment_ids.reshape(-1).astype(jnp.int32)
    out = _make_sc_call(ids.shape[0])(
        word_table, task_table, segment_table, tid, sid, ids)
    return out.reshape(shape + (D_MODEL,))


# speedup vs baseline: 4.3033x; 2.7489x over previous
import functools
import math

import jax
import jax.numpy as jnp
from jax import lax
from jax.experimental import pallas as pl
from jax.experimental.pallas import tpu as pltpu
from jax.experimental.pallas import tpu_sc as plsc

D_MODEL = 768
LANES = 16
NUM_CORES = 2
NUM_SUBCORES = 16
NUM_WORKERS = NUM_CORES * NUM_SUBCORES
CHUNK = 32
NBUF = 3
SCALE = 1.0 / math.sqrt(D_MODEL)


def _sc_body(word_hbm, task_hbm, seg_hbm, tid_hbm, sid_hbm, ids_hbm, out_hbm,
             small_v, comb_v, ids_vs, tid_vs, sid_vs, cidx_vs, bufs,
             isems, gsems, osems):
    n_tok = ids_hbm.shape[0]
    tok_per_w = n_tok // NUM_WORKERS
    nch = tok_per_w // CHUNK
    wid = lax.axis_index("s") * NUM_CORES + lax.axis_index("c")
    base = wid * tok_per_w

    pltpu.sync_copy(task_hbm, small_v.at[pl.ds(0, 3)])
    pltpu.sync_copy(seg_hbm, small_v.at[pl.ds(3, 3)])

    @pl.loop(0, D_MODEL // LANES)
    def _build(j):
        col = j * LANES
        for t in range(3):
            tv = small_v[t, pl.ds(col, LANES)]
            for s in range(3):
                sv = small_v[3 + s, pl.ds(col, LANES)]
                comb_v[pl.ds((t * 3 + s) * D_MODEL + col, LANES)] = tv + sv * SCALE

    lane_iota = lax.iota(jnp.int32, LANES)

    def issue_idx(k):
        b = k % NBUF
        off = base + k * CHUNK
        return (
            pltpu.async_copy(ids_hbm.at[pl.ds(off, CHUNK)], ids_vs[b], isems[b]),
            pltpu.async_copy(tid_hbm.at[pl.ds(off, CHUNK)], tid_vs[b], isems[b]),
            pltpu.async_copy(sid_hbm.at[pl.ds(off, CHUNK)], sid_vs[b], isems[b]),
        )

    def start_gather(k, idx_descs, out_desc_old):
        b = k % NBUF
        for d in idx_descs:
            d.wait()
        for g in range(CHUNK // LANES):
            sl = pl.ds(g * LANES, LANES)
            cidx_vs[b][sl] = tid_vs[b][sl] * 3 + sid_vs[b][sl]
        if out_desc_old is not None:
            out_desc_old.wait()
        return pltpu.async_copy(word_hbm.at[ids_vs[b]], bufs[b], gsems[b])

    def finish(k, gather_desc):
        b = k % NBUF
        gather_desc.wait()
        buf = bufs[b]
        cidx_v = cidx_vs[b]

        @pl.loop(0, CHUNK)
        def _add(t):
            ctv = plsc.load_gather(cidx_v, [jnp.full((LANES,), 0, jnp.int32) + t])
            cbase = ctv * D_MODEL + lane_iota
            nj = D_MODEL // LANES
            depth = 4
            pend = [plsc.load_gather(comb_v, [cbase + (j * LANES)])
                    for j in range(depth)]
            for j in range(nj):
                if j + depth < nj:
                    pend.append(
                        plsc.load_gather(comb_v, [cbase + ((j + depth) * LANES)]))
                plsc.addupdate(buf.at[t, pl.ds(j * LANES, LANES)], pend.pop(0))

        off = base + k * CHUNK
        return pltpu.async_copy(buf, out_hbm.at[pl.ds(off, CHUNK)], osems[b])

    idxd = {0: issue_idx(0)}
    if nch > 1:
        idxd[1] = issue_idx(1)
    outd = {}
    gd = {0: start_gather(0, idxd.pop(0), None)}
    for k in range(nch):
        if k + 1 < nch:
            gd[k + 1] = start_gather(k + 1, idxd.pop(k + 1),
                                     outd.get(k + 1 - NBUF))
        if k + 2 < nch:
            idxd[k + 2] = issue_idx(k + 2)
        outd[k] = finish(k, gd.pop(k))
    for k in range(max(0, nch - NBUF + 1), nch):
        outd[k].wait()


@functools.lru_cache(maxsize=None)
def _make_sc_call(n_tok: int):
    def body(word_hbm, task_hbm, seg_hbm, tid_hbm, sid_hbm, ids_hbm, out_hbm,
             small_v, comb_v,
             i0, i1, i2, t0, t1, t2, s0, s1, s2, c0, c1, c2, b0, b1, b2,
             is0, is1, is2, gs0, gs1, gs2, os0, os1, os2):
        _sc_body(word_hbm, task_hbm, seg_hbm, tid_hbm, sid_hbm, ids_hbm,
                 out_hbm, small_v, comb_v,
                 (i0, i1, i2), (t0, t1, t2), (s0, s1, s2), (c0, c1, c2),
                 (b0, b1, b2), (is0, is1, is2), (gs0, gs1, gs2),
                 (os0, os1, os2))

    idx = pltpu.VMEM((CHUNK,), jnp.int32)
    buf = pltpu.VMEM((CHUNK, D_MODEL), jnp.float32)
    sem = pltpu.SemaphoreType.DMA
    return pl.kernel(
        body,
        out_type=jax.ShapeDtypeStruct((n_tok, D_MODEL), jnp.float32),
        mesh=plsc.VectorSubcoreMesh(core_axis_name="c", subcore_axis_name="s"),
        compiler_params=pltpu.CompilerParams(needs_layout_passes=False),
        scratch_types=[
            pltpu.VMEM((6, D_MODEL), jnp.float32),
            pltpu.VMEM((9 * D_MODEL,), jnp.float32),
            idx, idx, idx,
            idx, idx, idx,
            idx, idx, idx,
            idx, idx, idx,
            buf, buf, buf,
            sem, sem, sem,
            sem, sem, sem,
            sem, sem, sem,
        ],
    )


@jax.jit
def kernel(input_ids, task_ids, segment_ids, word_table, task_table, segment_table):
    shape = input_ids.shape
    ids = input_ids.reshape(-1).astype(jnp.int32)
    tid = task_ids.reshape(-1).astype(jnp.int32)
    sid = segment_ids.reshape(-1).astype(jnp.int32)
    out = _make_sc_call(ids.shape[0])(
        word_table, task_table, segment_table, tid, sid, ids)
    return out.reshape(shape + (D_MODEL,))
